# baseline (speedup 1.0000x reference)
import jax
import jax.numpy as jnp
from jax import lax
from jax.experimental import pallas as pl
from jax.experimental.pallas import tpu as pltpu
from jax.experimental.pallas import tpu_sc as plsc

_FIELD = 26
_EMB = 32
_BATCH = 16384
_DENSE = 13
_NC, _NS = 2, 16
_NW = _NC * _NS
_RPW = _BATCH // _NW
_C = 32
_NCHUNK = _RPW // _C
_NIDX = _C * _FIELD
_NIDX_W = _RPW * _FIELD


def _fm_body(xs_hbm, xd_hbm, w_hbm, emb_hbm, par_hbm, out_hbm,
             idx_v, idxw_v, emb_v0, emb_v1, w_v0, w_v1, xd_v, out_v, par_v,
             sem0, sem1):
    wid = lax.axis_index("s") * _NC + lax.axis_index("c")
    base_row = wid * _RPW
    pltpu.sync_copy(par_hbm, par_v)
    pv = par_v[...]
    iota = lax.iota(jnp.int32, 16)

    pltpu.sync_copy(xs_hbm.at[pl.ds(base_row * _FIELD, _NIDX_W)], idx_v)
    pltpu.sync_copy(xd_hbm.at[pl.ds(base_row, _RPW), :], xd_v)

    def off_body(k, _):
        sl = pl.ds(k * 16, 16)
        pos = jnp.remainder(iota + k * 16, _FIELD) * 100000
        v = idx_v[sl] + pos
        idx_v[sl] = v
        idxw_v[sl] = lax.shift_right_logical(v, 4)
        return _

    emb_bufs = (emb_v0, emb_v1)
    w_bufs = (w_v0, w_v1)
    sems = (sem0, sem1)

    def fire(c):
        b = c % 2
        sl = pl.ds(c * _NIDX, _NIDX)
        ce = pltpu.async_copy(emb_hbm.at[idx_v.at[sl]], emb_bufs[b], sems[b])
        cw = pltpu.async_copy(w_hbm.at[idxw_v.at[sl]], w_bufs[b], sems[b])
        return ce, cw

    prep0 = _NIDX // 16
    lax.fori_loop(0, prep0, off_body, None)
    pend = {0: fire(0)}
    lax.fori_loop(prep0, _NIDX_W // 16, off_body, None)
    pend[1] = fire(1)

    for c in range(_NCHUNK):
        ce, cw = pend.pop(c)
        ce.wait()
        cw.wait()
        emb_v = emb_bufs[c % 2]
        w_v = w_bufs[c % 2]

        for g in range(_C // 16):
            rbase = (iota + g * 16) * _FIELD
            gbase = rbase + c * _NIDX
            lane = iota + (c * _C + g * 16)
            zero16 = jnp.zeros((16,), jnp.float32)
            zeroi = jnp.zeros((16,), jnp.int32)

            def f_body(f, fc, rbase=rbase, gbase=gbase, emb_v=emb_v, w_v=w_v):
                q, wacc = fc[0], fc[1]
                s = list(fc[2:])
                row = rbase + f
                col = zeroi + 0
                for dd in range(16):
                    v = plsc.load_gather(emb_v, [row, col])
                    if dd + 1 < 16:
                        col = col + 1
                    lo = plsc.bitcast(lax.shift_left(v, 16), jnp.float32)
                    hi = plsc.bitcast(jnp.bitwise_and(v, -65536), jnp.float32)
                    s[2 * dd] = s[2 * dd] + lo
                    s[2 * dd + 1] = s[2 * dd + 1] + hi
                    q = q + lo * lo + hi * hi
                wcol = jnp.bitwise_and(plsc.load_gather(idx_v, [gbase + f]), 15)
                wacc = wacc + plsc.load_gather(w_v, [row, wcol])
                return tuple([q, wacc] + s)

            res = lax.fori_loop(0, _FIELD, f_body, tuple([zero16] * (2 + _EMB)))
            q, wacc, s = res[0], res[1], res[2:]
            inter = s[0] * s[0]
            for d in range(1, _EMB):
                inter = inter + s[d] * s[d]
            inter = 0.5 * (inter - q)

            lin = jnp.full((16,), pv[13] + pv[14], jnp.float32)
            dcol = zeroi + 0
            for j in range(_DENSE):
                xv = plsc.load_gather(xd_v, [lane, dcol])
                if j + 1 < _DENSE:
                    dcol = dcol + 1
                lin = lin + xv * pv[j]

            out_v[pl.ds(c * _C + g * 16, 16)] = inter + wacc + lin

        if c + 2 < _NCHUNK:
            pend[c + 2] = fire(c + 2)

    pltpu.sync_copy(out_v, out_hbm.at[pl.ds(base_row, _RPW)])


def kernel(x_dense, x_sparse, w_lin, b_lin, emb_table, W_dense, b_dense):
    xs_flat = x_sparse.reshape(-1).astype(jnp.int32)
    emb_i = jax.lax.bitcast_convert_type(
        emb_table.astype(jnp.bfloat16).reshape(-1, _EMB // 2, 2), jnp.int32)
    params = jnp.concatenate(
        [W_dense.reshape(-1).astype(jnp.float32),
         b_lin.reshape(-1).astype(jnp.float32),
         b_dense.reshape(-1).astype(jnp.float32),
         jnp.zeros((1,), jnp.float32)])

    mesh = plsc.VectorSubcoreMesh(core_axis_name="c", subcore_axis_name="s",
                                  num_cores=_NC, num_subcores=_NS)
    out = pl.kernel(
        _fm_body,
        out_type=jax.ShapeDtypeStruct((_BATCH,), jnp.float32),
        mesh=mesh,
        scratch_types=[
            pltpu.VMEM((_NIDX_W,), jnp.int32),
            pltpu.VMEM((_NIDX_W,), jnp.int32),
            pltpu.VMEM((_NIDX, _EMB // 2), jnp.int32),
            pltpu.VMEM((_NIDX, _EMB // 2), jnp.int32),
            pltpu.VMEM((_NIDX, 16), jnp.float32),
            pltpu.VMEM((_NIDX, 16), jnp.float32),
            pltpu.VMEM((_RPW, _DENSE), jnp.float32),
            pltpu.VMEM((_RPW,), jnp.float32),
            pltpu.VMEM((16,), jnp.float32),
            pltpu.SemaphoreType.DMA,
            pltpu.SemaphoreType.DMA,
        ],
        compiler_params=pltpu.CompilerParams(needs_layout_passes=False,
                                             use_tc_tiling_on_sc=False),
    )(xs_flat, x_dense, w_lin.reshape(-1, 16), emb_i, params)
    return out.reshape(_BATCH, 1)


# speedup vs baseline: 1.9362x; 1.9362x over previous
import jax
import jax.numpy as jnp
from jax import lax
from jax.experimental import pallas as pl
from jax.experimental.pallas import tpu as pltpu
from jax.experimental.pallas import tpu_sc as plsc

_FIELD = 26
_EMB = 32
_BATCH = 16384
_DENSE = 13
_NC, _NS = 2, 16
_NW = _NC * _NS
_RPW = _BATCH // _NW
_C = 32
_NCHUNK = _RPW // _C
_NIDX = _C * _FIELD
_NIDX_W = _RPW * _FIELD


def _fm_body(xs_hbm, xd_hbm, w_hbm, emb_hbm, par_hbm, out_hbm,
             idx2_v, idxw_v, emb_v0, emb_v1, w_v0, w_v1, xd_v, out_v, par_v,
             sem0, sem1):
    wid = lax.axis_index("s") * _NC + lax.axis_index("c")
    base_row = wid * _RPW
    pltpu.sync_copy(par_hbm, par_v)
    pv = par_v[...]
    iota = lax.iota(jnp.int32, 16)

    pltpu.sync_copy(xs_hbm.at[pl.ds(base_row * _FIELD, _NIDX_W)], idxw_v)
    pltpu.sync_copy(xd_hbm.at[pl.ds(base_row, _RPW), :], xd_v)

    def off_body(k, _):
        sl = pl.ds(k * 16, 16)
        pos = jnp.remainder(iota + k * 16, _FIELD) * 100000
        v = idxw_v[sl] + pos
        v2 = v * 2
        dpos = (iota + k * 16) * 2
        plsc.store_scatter(idx2_v, [dpos], v2)
        plsc.store_scatter(idx2_v, [dpos + 1], v2 + 1)
        idxw_v[sl] = lax.shift_right_logical(v, 4)
        return _

    emb_bufs = (emb_v0, emb_v1)
    w_bufs = (w_v0, w_v1)
    sems = (sem0, sem1)

    def fire(c):
        b = c % 2
        ce = pltpu.async_copy(emb_hbm.at[idx2_v.at[pl.ds(c * _NIDX * 2, _NIDX * 2)]],
                              emb_bufs[b], sems[b])
        cw = pltpu.async_copy(w_hbm.at[idxw_v.at[pl.ds(c * _NIDX, _NIDX)]],
                              w_bufs[b], sems[b])
        return ce, cw

    prep0 = _NIDX // 16
    lax.fori_loop(0, prep0, off_body, None)
    pend = {0: fire(0)}
    lax.fori_loop(prep0, _NIDX_W // 16, off_body, None)
    pend[1] = fire(1)

    for c in range(_NCHUNK):
        b = c % 2
        ce, cw = pend.pop(c)
        ce.wait()
        cw.wait()
        emb_v = emb_bufs[b]
        w_v = w_bufs[b]

        for g in range(_C // 16):
            rbase = (iota + g * 16) * _FIELD
            r2base = rbase * 2
            gbase2 = (rbase + c * _NIDX) * 2
            lane = iota + (c * _C + g * 16)
            zero16 = jnp.zeros((16,), jnp.float32)

            zeroi = jnp.zeros((16,), jnp.int32)

            def f_body(f, fc, r2base=r2base, gbase2=gbase2, emb_v=emb_v, w_v=w_v):
                q, wacc = fc[0], fc[1]
                s = list(fc[2:])
                row2 = r2base + f * 2
                col = zeroi + 0
                for d in range(16):
                    v = plsc.load_gather(emb_v, [row2, col])
                    if d + 1 < 16:
                        col = col + 1
                    s[d] = s[d] + v
                    q = q + v * v
                row2b = row2 + 1
                col = zeroi + 0
                for d in range(16, _EMB):
                    v = plsc.load_gather(emb_v, [row2b, col])
                    if d + 1 < _EMB:
                        col = col + 1
                    s[d] = s[d] + v
                    q = q + v * v
                v2 = plsc.load_gather(idx2_v, [gbase2 + f * 2])
                wcol = jnp.bitwise_and(lax.shift_right_logical(v2, 1), 15)
                wrow = lax.shift_right_logical(row2, 1)
                wacc = wacc + plsc.load_gather(w_v, [wrow, wcol])
                return tuple([q, wacc] + s)

            res = lax.fori_loop(0, _FIELD, f_body, tuple([zero16] * (2 + _EMB)))
            q, wacc, s = res[0], res[1], res[2:]
            inter = s[0] * s[0]
            for d in range(1, _EMB):
                inter = inter + s[d] * s[d]
            inter = 0.5 * (inter - q)

            lin = jnp.full((16,), pv[13] + pv[14], jnp.float32)
            dcol = zeroi + 0
            for j in range(_DENSE):
                xv = plsc.load_gather(xd_v, [lane, dcol])
                if j + 1 < _DENSE:
                    dcol = dcol + 1
                lin = lin + xv * pv[j]

            out_v[pl.ds(c * _C + g * 16, 16)] = inter + wacc + lin

        if c + 2 < _NCHUNK:
            pend[c + 2] = fire(c + 2)

    pltpu.sync_copy(out_v, out_hbm.at[pl.ds(base_row, _RPW)])


def kernel(x_dense, x_sparse, w_lin, b_lin, emb_table, W_dense, b_dense):
    xs_flat = x_sparse.reshape(-1).astype(jnp.int32)
    params = jnp.concatenate(
        [W_dense.reshape(-1).astype(jnp.float32),
         b_lin.reshape(-1).astype(jnp.float32),
         b_dense.reshape(-1).astype(jnp.float32),
         jnp.zeros((1,), jnp.float32)])

    mesh = plsc.VectorSubcoreMesh(core_axis_name="c", subcore_axis_name="s",
                                  num_cores=_NC, num_subcores=_NS)
    out = pl.kernel(
        _fm_body,
        out_type=jax.ShapeDtypeStruct((_BATCH,), jnp.float32),
        mesh=mesh,
        scratch_types=[
            pltpu.VMEM((_NIDX_W * 2,), jnp.int32),
            pltpu.VMEM((_NIDX_W,), jnp.int32),
            pltpu.VMEM((_NIDX * 2, 16), jnp.float32),
            pltpu.VMEM((_NIDX * 2, 16), jnp.float32),
            pltpu.VMEM((_NIDX, 16), jnp.float32),
            pltpu.VMEM((_NIDX, 16), jnp.float32),
            pltpu.VMEM((_RPW, _DENSE), jnp.float32),
            pltpu.VMEM((_RPW,), jnp.float32),
            pltpu.VMEM((16,), jnp.float32),
            pltpu.SemaphoreType.DMA,
            pltpu.SemaphoreType.DMA,
        ],
        compiler_params=pltpu.CompilerParams(needs_layout_passes=False,
                                             use_tc_tiling_on_sc=False),
    )(xs_flat, x_dense, w_lin.reshape(-1, 16), emb_table.reshape(-1, 16), params)
    return out.reshape(_BATCH, 1)
